# Initial kernel scaffold; baseline (speedup 1.0000x reference)
import functools
import jax, jax.numpy as jnp
from jax import lax
from jax.experimental import pallas as pl
from jax.experimental.pallas import tpu as pltpu, tpu_sc as plsc

N, E, NG, F, EF, EH = 10000, 160000, 64, 24, 16, 32
E_PAD = 163840
NW = 32
EPW = E_PAD // NW
CHUNK = 128
NCHUNK = EPW // CHUNK
EB = 2048

_mesh = plsc.VectorSubcoreMesh(core_axis_name="c", subcore_axis_name="s")


def _gather_body(x_hbm, src_hbm, out_hbm, idx_v, rows_v, sem):
    wid = lax.axis_index("s") * 2 + lax.axis_index("c")
    base = wid * EPW
    pltpu.sync_copy(src_hbm.at[pl.ds(base, EPW)], idx_v)

    def body(j, carry):
        off = j * CHUNK
        pltpu.async_copy(x_hbm.at[idx_v.at[pl.ds(off, CHUNK)]], rows_v, sem).wait()
        pltpu.sync_copy(rows_v, out_hbm.at[pl.ds(base + off, CHUNK)])
        return carry

    lax.fori_loop(0, NCHUNK, body, 0)


def _sc_gather(x32, src_pad):
    return pl.kernel(
        _gather_body,
        out_type=jax.ShapeDtypeStruct((E_PAD, 32), jnp.float32),
        mesh=_mesh,
        scratch_types=[
            pltpu.VMEM((EPW,), jnp.int32),
            pltpu.VMEM((CHUNK, 32), jnp.float32),
            pltpu.SemaphoreType.DMA,
        ],
    )(x32, src_pad)


def _scatter_body(msg_hbm, dst3_hbm, zeros_hbm, out_hbm, idx_v, rows_v, accum, sem):
    c = lax.axis_index("c")
    s = lax.axis_index("s")
    wid = s * 2 + c
    stripe = N // 16
    pltpu.sync_copy(zeros_hbm.at[pl.ds(s * stripe, stripe)],
                    accum.at[pl.ds(s * stripe, stripe)])
    pltpu.sync_copy(dst3_hbm.at[wid], idx_v)
    plsc.subcore_barrier()

    def body(j, carry):
        pltpu.sync_copy(msg_hbm.at[pl.ds(wid * EPW + j * CHUNK, CHUNK)], rows_v)
        pltpu.sync_copy(rows_v, accum.at[idx_v.at[j]], add=True)
        return carry

    lax.fori_loop(0, NCHUNK, body, 0)
    plsc.subcore_barrier()
    pltpu.sync_copy(accum.at[pl.ds(s * stripe, stripe)],
                    out_hbm.at[c, pl.ds(s * stripe, stripe)])


def _sc_scatter(msg, dst3, zeros):
    return pl.kernel(
        _scatter_body,
        out_type=jax.ShapeDtypeStruct((2, N, 32), jnp.float32),
        mesh=_mesh,
        scratch_types=[
            pltpu.VMEM((NCHUNK, CHUNK), jnp.int32),
            pltpu.VMEM((CHUNK, 32), jnp.float32),
            pltpu.VMEM_SHARED((N, 32), jnp.float32),
            pltpu.SemaphoreType.DMA,
        ],
    )(msg, dst3, zeros)


def _edge_body(ea_ref, xj_ref, en1_wT, en1_b, en2_wT, rp32, s32, b232, out_ref):
    ea = ea_ref[...]
    xj = xj_ref[...]
    h = jnp.maximum(ea @ en1_wT[...] + en1_b[...], 0.0)
    w = h @ en2_wT[...]
    p = (xj @ rp32[...]) * w
    msg = p @ s32[...] + xj @ b232[...]
    lane = lax.broadcasted_iota(jnp.int32, (EB, 32), 1)
    msg = msg + jnp.where(lane == 24, 1.0, 0.0)
    row = lax.broadcasted_iota(jnp.int32, (EB, 32), 0) + pl.program_id(0) * EB
    out_ref[...] = jnp.where(row < E, msg, 0.0)


def _edge_msg(ea_pad, xj, en1_wT, en1_b, en2_wT, rp32, s32, b232):
    return pl.pallas_call(
        _edge_body,
        grid=(E_PAD // EB,),
        in_specs=[
            pl.BlockSpec((EB, EF), lambda i: (i, 0)),
            pl.BlockSpec((EB, 32), lambda i: (i, 0)),
            pl.BlockSpec((EF, EH), lambda i: (0, 0)),
            pl.BlockSpec((1, EH), lambda i: (0, 0)),
            pl.BlockSpec((EH, 576), lambda i: (0, 0)),
            pl.BlockSpec((32, 576), lambda i: (0, 0)),
            pl.BlockSpec((576, 32), lambda i: (0, 0)),
            pl.BlockSpec((32, 32), lambda i: (0, 0)),
        ],
        out_specs=pl.BlockSpec((EB, 32), lambda i: (i, 0)),
        out_shape=jax.ShapeDtypeStruct((E_PAD, 32), jnp.float32),
    )(ea_pad, xj, en1_wT, en1_b, en2_wT, rp32, s32, b232)


def _node_body(do_relu, parts_ref, x_ref, rw_ref, cb_ref, out_ref):
    s = parts_ref[0] + parts_ref[1]
    cnt = jnp.maximum(s[:, 24:25], 1.0)
    y = s / cnt + x_ref[...] @ rw_ref[...] + cb_ref[...]
    lane = lax.broadcasted_iota(jnp.int32, (N, 32), 1)
    y = jnp.where(lane < 24, y, 0.0)
    if do_relu:
        y = jnp.maximum(y, 0.0)
    out_ref[...] = y


def _node_update(parts, x32, rw32, cb32, do_relu):
    return pl.pallas_call(
        functools.partial(_node_body, do_relu),
        out_shape=jax.ShapeDtypeStruct((N, 32), jnp.float32),
    )(parts, x32, rw32, cb32)


def _s2s_body(parts_ref, x1_ref, rw_ref, cb_ref, batch_ref,
              wih_ref, whh_ref, bb_ref, fc2_ref, fc2b_ref, fc3_ref, fc3b_ref,
              out_ref):
    s = parts_ref[0] + parts_ref[1]
    cnt = jnp.maximum(s[:, 24:25], 1.0)
    lane = lax.broadcasted_iota(jnp.int32, (N, 32), 1)
    x2 = s / cnt + x1_ref[...] @ rw_ref[...] + cb_ref[...]
    x2 = jnp.where(lane < 24, x2, 0.0)

    at = (batch_ref[...] == lax.broadcasted_iota(jnp.int32, (N, NG), 1)).astype(jnp.float32)
    h = jnp.zeros((NG, 32), jnp.float32)
    c = jnp.zeros((NG, 32), jnp.float32)
    qs = jnp.zeros((NG, 64), jnp.float32)
    wih = wih_ref[...]
    whh = whh_ref[...]
    bb = bb_ref[...]
    for _ in range(3):
        g = qs @ wih + h @ whh + bb
        gi = jax.nn.sigmoid(g[:, 0:32])
        gf = jax.nn.sigmoid(g[:, 32:64])
        gg = jnp.tanh(g[:, 64:96])
        go = jax.nn.sigmoid(g[:, 96:128])
        c = gf * c + gi * gg
        h = go * jnp.tanh(c)
        glane = lax.broadcasted_iota(jnp.int32, (NG, 32), 1)
        q = jnp.where(glane < 24, h, 0.0)
        e = jnp.sum(x2 * (at @ q), axis=1, keepdims=True)
        e2 = jnp.where(at > 0.0, e, -1e30)
        m = jnp.max(e2, axis=0, keepdims=True)
        m_g = lax.dot_general(at, m, (((1,), (1,)), ((), ())))
        a = jnp.exp(e - m_g)
        asum = lax.dot_general(a, at, (((0,), (0,)), ((), ())))
        asum_g = lax.dot_general(at, asum, (((1,), (1,)), ((), ())))
        a = a / (asum_g + 1e-16)
        r = lax.dot_general(at, a * x2, (((0,), (0,)), ((), ())))
        qs = jnp.concatenate([q, r], axis=1)
    z = jnp.maximum(qs @ fc2_ref[...] + fc2b_ref[...], 0.0)
    out_ref[...] = z @ fc3_ref[...] + fc3b_ref[...]


def _set2set(parts2, x1, rw32, cb32, batch_col, wih_p, whh_p, bb_p, fc2_p, fc2b, fc3_p, fc3b):
    return pl.pallas_call(
        _s2s_body,
        out_shape=jax.ShapeDtypeStruct((NG, 2), jnp.float32),
    )(parts2, x1, rw32, cb32, batch_col, wih_p, whh_p, bb_p, fc2_p, fc2b, fc3_p, fc3b)


def _prep_consts(en1_w, en2_w, en2_b, root_w, conv_b, lstm_wih, lstm_whh,
                 lstm_bih, lstm_bhh, fc2_w, fc2_b, fc3_w, fc3_b):
    rp32 = jnp.zeros((32, 576)).at[:F].set(jnp.repeat(jnp.eye(F), F, axis=1))
    s32 = jnp.zeros((576, 32)).at[:, :F].set(jnp.tile(jnp.eye(F), (F, 1)))
    b232 = jnp.zeros((32, 32)).at[:F, :F].set(en2_b.reshape(F, F))
    rw32 = jnp.zeros((32, 32)).at[:F, :F].set(root_w)
    cb32 = jnp.zeros((1, 32)).at[0, :F].set(conv_b)
    wih_p = jnp.zeros((64, 128))
    whh_p = jnp.zeros((32, 128))
    bb_p = jnp.zeros((1, 128))
    for gi in range(4):
        wg = lstm_wih[gi * F:(gi + 1) * F]
        wih_p = wih_p.at[0:F, gi * 32:gi * 32 + F].set(wg[:, :F].T)
        wih_p = wih_p.at[32:32 + F, gi * 32:gi * 32 + F].set(wg[:, F:].T)
        whh_p = whh_p.at[0:F, gi * 32:gi * 32 + F].set(lstm_whh[gi * F:(gi + 1) * F].T)
        bb_p = bb_p.at[0, gi * 32:gi * 32 + F].set(
            lstm_bih[gi * F:(gi + 1) * F] + lstm_bhh[gi * F:(gi + 1) * F])
    fc2_p = jnp.zeros((64, 8)).at[0:F].set(fc2_w[:, :F].T).at[32:32 + F].set(fc2_w[:, F:].T)
    return dict(en1_wT=en1_w.T, en2_wT=en2_w.T, rp32=rp32, s32=s32, b232=b232,
                rw32=rw32, cb32=cb32, wih_p=wih_p, whh_p=whh_p, bb_p=bb_p,
                fc2_p=fc2_p, fc2b=fc2_b.reshape(1, 8), fc3_p=fc3_w.T,
                fc3b=fc3_b.reshape(1, 2))


def kernel(x, edge_index, edge_attr, batch, en1_w, en1_b, en2_w, en2_b, root_w, conv_b,
           lstm_wih, lstm_whh, lstm_bih, lstm_bhh, fc2_w, fc2_b, fc3_w, fc3_b):
    C = _prep_consts(en1_w, en2_w, en2_b, root_w, conv_b, lstm_wih, lstm_whh,
                     lstm_bih, lstm_bhh, fc2_w, fc2_b, fc3_w, fc3_b)
    en1_b_r = en1_b.reshape(1, EH)
    src = jnp.pad(edge_index[0], (0, E_PAD - E))
    dst3 = jnp.pad(edge_index[1], (0, E_PAD - E)).reshape(NW, NCHUNK, CHUNK)
    ea_pad = jnp.pad(edge_attr, ((0, E_PAD - E), (0, 0)))
    x32 = jnp.pad(x, ((0, 0), (0, 8)))
    batch_col = batch.reshape(N, 1)
    zeros = jnp.zeros((N, 32), jnp.float32)

    def layer(xc):
        xj = _sc_gather(xc, src)
        msg = _edge_msg(ea_pad, xj, C["en1_wT"], en1_b_r, C["en2_wT"],
                        C["rp32"], C["s32"], C["b232"])
        return _sc_scatter(msg, dst3, zeros)

    parts1 = layer(x32)
    x1 = _node_update(parts1, x32, C["rw32"], C["cb32"], True)
    parts2 = layer(x1)
    return _set2set(parts2, x1, C["rw32"], C["cb32"], batch_col,
                    C["wih_p"], C["whh_p"], C["bb_p"], C["fc2_p"], C["fc2b"],
                    C["fc3_p"], C["fc3b"])


# speedup vs baseline: 3.5398x; 3.5398x over previous
import functools
import jax, jax.numpy as jnp
from jax import lax
from jax.experimental import pallas as pl
from jax.experimental.pallas import tpu as pltpu, tpu_sc as plsc

N, E, NG, F, EF, EH = 10000, 160000, 64, 24, 16, 32
E_PAD = 163840
NW = 32
EPW = E_PAD // NW
CHUNK = 128
NCHUNK = EPW // CHUNK
EB = 2048

_mesh = plsc.VectorSubcoreMesh(core_axis_name="c", subcore_axis_name="s")


def _gather_body(x_hbm, src_hbm, out_hbm, idx_v, rows_v, sem):
    wid = lax.axis_index("s") * 2 + lax.axis_index("c")
    base = wid * EPW
    pltpu.sync_copy(src_hbm.at[pl.ds(base, EPW)], idx_v)

    def body(j, carry):
        off = j * CHUNK
        pltpu.async_copy(x_hbm.at[idx_v.at[pl.ds(off, CHUNK)]], rows_v, sem).wait()
        pltpu.sync_copy(rows_v, out_hbm.at[pl.ds(base + off, CHUNK)])
        return carry

    lax.fori_loop(0, NCHUNK, body, 0)


def _sc_gather(x32, src_pad):
    return pl.kernel(
        _gather_body,
        out_type=jax.ShapeDtypeStruct((E_PAD, 32), jnp.float32),
        mesh=_mesh,
        compiler_params=pltpu.CompilerParams(use_tc_tiling_on_sc=False),
        scratch_types=[
            pltpu.VMEM((EPW,), jnp.int32),
            pltpu.VMEM((CHUNK, 32), jnp.float32),
            pltpu.SemaphoreType.DMA,
        ],
    )(x32, src_pad)


def _scatter_body(msg_hbm, dst3_hbm, zeros_hbm, out_hbm, idx_v, rows_v, accum, sem):
    c = lax.axis_index("c")
    s = lax.axis_index("s")
    wid = s * 2 + c
    stripe = N // 16
    pltpu.sync_copy(zeros_hbm.at[pl.ds(s * stripe, stripe)],
                    accum.at[pl.ds(s * stripe, stripe)])
    pltpu.sync_copy(dst3_hbm.at[wid], idx_v)
    plsc.subcore_barrier()

    def body(j, carry):
        pltpu.sync_copy(msg_hbm.at[pl.ds(wid * EPW + j * CHUNK, CHUNK)], rows_v)
        pltpu.sync_copy(rows_v, accum.at[idx_v.at[j]], add=True)
        return carry

    lax.fori_loop(0, NCHUNK, body, 0)
    plsc.subcore_barrier()
    pltpu.sync_copy(accum.at[pl.ds(s * stripe, stripe)],
                    out_hbm.at[c, pl.ds(s * stripe, stripe)])


def _sc_scatter(msg, dst3, zeros):
    return pl.kernel(
        _scatter_body,
        out_type=jax.ShapeDtypeStruct((2, N, 32), jnp.float32),
        mesh=_mesh,
        compiler_params=pltpu.CompilerParams(use_tc_tiling_on_sc=False),
        scratch_types=[
            pltpu.VMEM((NCHUNK, CHUNK), jnp.int32),
            pltpu.VMEM((CHUNK, 32), jnp.float32),
            pltpu.VMEM_SHARED((N, 32), jnp.float32),
            pltpu.SemaphoreType.DMA,
        ],
    )(msg, dst3, zeros)


def _edge_body(ea_ref, xj_ref, en1_wT, en1_b, en2_wT, rp32, s32, b232, out_ref):
    ea = ea_ref[...]
    xj = xj_ref[...]
    h = jnp.maximum(ea @ en1_wT[...] + en1_b[...], 0.0)
    w = h @ en2_wT[...]
    p = (xj @ rp32[...]) * w
    msg = p @ s32[...] + xj @ b232[...]
    lane = lax.broadcasted_iota(jnp.int32, (EB, 32), 1)
    msg = msg + jnp.where(lane == 24, 1.0, 0.0)
    row = lax.broadcasted_iota(jnp.int32, (EB, 32), 0) + pl.program_id(0) * EB
    out_ref[...] = jnp.where(row < E, msg, 0.0)


def _edge_msg(ea_pad, xj, en1_wT, en1_b, en2_wT, rp32, s32, b232):
    return pl.pallas_call(
        _edge_body,
        grid=(E_PAD // EB,),
        in_specs=[
            pl.BlockSpec((EB, EF), lambda i: (i, 0)),
            pl.BlockSpec((EB, 32), lambda i: (i, 0)),
            pl.BlockSpec((EF, EH), lambda i: (0, 0)),
            pl.BlockSpec((1, EH), lambda i: (0, 0)),
            pl.BlockSpec((EH, 576), lambda i: (0, 0)),
            pl.BlockSpec((32, 576), lambda i: (0, 0)),
            pl.BlockSpec((576, 32), lambda i: (0, 0)),
            pl.BlockSpec((32, 32), lambda i: (0, 0)),
        ],
        out_specs=pl.BlockSpec((EB, 32), lambda i: (i, 0)),
        out_shape=jax.ShapeDtypeStruct((E_PAD, 32), jnp.float32),
    )(ea_pad, xj, en1_wT, en1_b, en2_wT, rp32, s32, b232)


def _node_body(do_relu, parts_ref, x_ref, rw_ref, cb_ref, out_ref):
    s = parts_ref[0] + parts_ref[1]
    cnt = jnp.maximum(s[:, 24:25], 1.0)
    y = s / cnt + x_ref[...] @ rw_ref[...] + cb_ref[...]
    lane = lax.broadcasted_iota(jnp.int32, (N, 32), 1)
    y = jnp.where(lane < 24, y, 0.0)
    if do_relu:
        y = jnp.maximum(y, 0.0)
    out_ref[...] = y


def _node_update(parts, x32, rw32, cb32, do_relu):
    return pl.pallas_call(
        functools.partial(_node_body, do_relu),
        out_shape=jax.ShapeDtypeStruct((N, 32), jnp.float32),
    )(parts, x32, rw32, cb32)


def _s2s_body(parts_ref, x1_ref, rw_ref, cb_ref, batch_ref,
              wih_ref, whh_ref, bb_ref, fc2_ref, fc2b_ref, fc3_ref, fc3b_ref,
              out_ref):
    s = parts_ref[0] + parts_ref[1]
    cnt = jnp.maximum(s[:, 24:25], 1.0)
    lane = lax.broadcasted_iota(jnp.int32, (N, 32), 1)
    x2 = s / cnt + x1_ref[...] @ rw_ref[...] + cb_ref[...]
    x2 = jnp.where(lane < 24, x2, 0.0)

    at = (batch_ref[...] == lax.broadcasted_iota(jnp.int32, (N, NG), 1)).astype(jnp.float32)
    h = jnp.zeros((NG, 32), jnp.float32)
    c = jnp.zeros((NG, 32), jnp.float32)
    qs = jnp.zeros((NG, 64), jnp.float32)
    wih = wih_ref[...]
    whh = whh_ref[...]
    bb = bb_ref[...]
    for _ in range(3):
        g = qs @ wih + h @ whh + bb
        gi = jax.nn.sigmoid(g[:, 0:32])
        gf = jax.nn.sigmoid(g[:, 32:64])
        gg = jnp.tanh(g[:, 64:96])
        go = jax.nn.sigmoid(g[:, 96:128])
        c = gf * c + gi * gg
        h = go * jnp.tanh(c)
        glane = lax.broadcasted_iota(jnp.int32, (NG, 32), 1)
        q = jnp.where(glane < 24, h, 0.0)
        e = jnp.sum(x2 * (at @ q), axis=1, keepdims=True)
        e2 = jnp.where(at > 0.0, e, -1e30)
        m = jnp.max(e2, axis=0, keepdims=True)
        m_g = jnp.sum(at * m, axis=1, keepdims=True)
        a = jnp.exp(e - m_g)
        asum = lax.dot_general(a, at, (((0,), (0,)), ((), ())))
        asum_g = jnp.sum(at * asum, axis=1, keepdims=True)
        a = a / (asum_g + 1e-16)
        r = lax.dot_general(at, a * x2, (((0,), (0,)), ((), ())))
        qs = jnp.concatenate([q, r], axis=1)
    z = jnp.maximum(qs @ fc2_ref[...] + fc2b_ref[...], 0.0)
    out_ref[...] = z @ fc3_ref[...] + fc3b_ref[...]


def _set2set(parts2, x1, rw32, cb32, batch_col, wih_p, whh_p, bb_p, fc2_p, fc2b, fc3_p, fc3b):
    return pl.pallas_call(
        _s2s_body,
        out_shape=jax.ShapeDtypeStruct((NG, 2), jnp.float32),
    )(parts2, x1, rw32, cb32, batch_col, wih_p, whh_p, bb_p, fc2_p, fc2b, fc3_p, fc3b)


def _prep_consts(en1_w, en2_w, en2_b, root_w, conv_b, lstm_wih, lstm_whh,
                 lstm_bih, lstm_bhh, fc2_w, fc2_b, fc3_w, fc3_b):
    rp32 = jnp.zeros((32, 576)).at[:F].set(jnp.repeat(jnp.eye(F), F, axis=1))
    s32 = jnp.zeros((576, 32)).at[:, :F].set(jnp.tile(jnp.eye(F), (F, 1)))
    b232 = jnp.zeros((32, 32)).at[:F, :F].set(en2_b.reshape(F, F))
    rw32 = jnp.zeros((32, 32)).at[:F, :F].set(root_w)
    cb32 = jnp.zeros((1, 32)).at[0, :F].set(conv_b)
    wih_p = jnp.zeros((64, 128))
    whh_p = jnp.zeros((32, 128))
    bb_p = jnp.zeros((1, 128))
    for gi in range(4):
        wg = lstm_wih[gi * F:(gi + 1) * F]
        wih_p = wih_p.at[0:F, gi * 32:gi * 32 + F].set(wg[:, :F].T)
        wih_p = wih_p.at[32:32 + F, gi * 32:gi * 32 + F].set(wg[:, F:].T)
        whh_p = whh_p.at[0:F, gi * 32:gi * 32 + F].set(lstm_whh[gi * F:(gi + 1) * F].T)
        bb_p = bb_p.at[0, gi * 32:gi * 32 + F].set(
            lstm_bih[gi * F:(gi + 1) * F] + lstm_bhh[gi * F:(gi + 1) * F])
    fc2_p = jnp.zeros((64, 8)).at[0:F].set(fc2_w[:, :F].T).at[32:32 + F].set(fc2_w[:, F:].T)
    return dict(en1_wT=en1_w.T, en2_wT=en2_w.T, rp32=rp32, s32=s32, b232=b232,
                rw32=rw32, cb32=cb32, wih_p=wih_p, whh_p=whh_p, bb_p=bb_p,
                fc2_p=fc2_p, fc2b=fc2_b.reshape(1, 8), fc3_p=fc3_w.T,
                fc3b=fc3_b.reshape(1, 2))


def kernel(x, edge_index, edge_attr, batch, en1_w, en1_b, en2_w, en2_b, root_w, conv_b,
           lstm_wih, lstm_whh, lstm_bih, lstm_bhh, fc2_w, fc2_b, fc3_w, fc3_b):
    C = _prep_consts(en1_w, en2_w, en2_b, root_w, conv_b, lstm_wih, lstm_whh,
                     lstm_bih, lstm_bhh, fc2_w, fc2_b, fc3_w, fc3_b)
    en1_b_r = en1_b.reshape(1, EH)
    src = jnp.pad(edge_index[0], (0, E_PAD - E))
    dst3 = jnp.pad(edge_index[1], (0, E_PAD - E)).reshape(NW, NCHUNK, CHUNK)
    ea_pad = jnp.pad(edge_attr, ((0, E_PAD - E), (0, 0)))
    x32 = jnp.pad(x, ((0, 0), (0, 8)))
    batch_col = batch.reshape(N, 1)
    zeros = jnp.zeros((N, 32), jnp.float32)

    def layer(xc):
        xj = _sc_gather(xc, src)
        msg = _edge_msg(ea_pad, xj, C["en1_wT"], en1_b_r, C["en2_wT"],
                        C["rp32"], C["s32"], C["b232"])
        return _sc_scatter(msg, dst3, zeros)

    parts1 = layer(x32)
    x1 = _node_update(parts1, x32, C["rw32"], C["cb32"], True)
    parts2 = layer(x1)
    return _set2set(parts2, x1, C["rw32"], C["cb32"], batch_col,
                    C["wih_p"], C["whh_p"], C["bb_p"], C["fc2_p"], C["fc2b"],
                    C["fc3_p"], C["fc3b"])


# speedup vs baseline: 3.7573x; 1.0614x over previous
import functools
import jax, jax.numpy as jnp
from jax import lax
from jax.experimental import pallas as pl
from jax.experimental.pallas import tpu as pltpu, tpu_sc as plsc

N, E, NG, F, EF, EH = 10000, 160000, 64, 24, 16, 32
E_PAD = 163840
NW = 32
EPW = E_PAD // NW
CHUNK = 128
NCHUNK = EPW // CHUNK
EB = 2048

_mesh = plsc.VectorSubcoreMesh(core_axis_name="c", subcore_axis_name="s")


KG = 8
GROUP = KG * CHUNK
NGROUP = EPW // GROUP


def _gather_body(x_hbm, src_hbm, out_hbm, idx_v, rows_v, sem):
    wid = lax.axis_index("s") * 2 + lax.axis_index("c")
    base = wid * EPW
    pltpu.sync_copy(src_hbm.at[pl.ds(base, EPW)], idx_v)

    def body(g, carry):
        off = g * GROUP
        cps = []
        for b in range(KG):
            cps.append(pltpu.async_copy(
                x_hbm.at[idx_v.at[pl.ds(off + b * CHUNK, CHUNK)]],
                rows_v.at[pl.ds(b * CHUNK, CHUNK)], sem))
        for cp in cps:
            cp.wait()
        pltpu.sync_copy(rows_v, out_hbm.at[pl.ds(base + off, GROUP)])
        return carry

    lax.fori_loop(0, NGROUP, body, 0)


def _sc_gather(x32, src_pad):
    return pl.kernel(
        _gather_body,
        out_type=jax.ShapeDtypeStruct((E_PAD, 32), jnp.float32),
        mesh=_mesh,
        compiler_params=pltpu.CompilerParams(use_tc_tiling_on_sc=False),
        scratch_types=[
            pltpu.VMEM((EPW,), jnp.int32),
            pltpu.VMEM((GROUP, 32), jnp.float32),
            pltpu.SemaphoreType.DMA,
        ],
    )(x32, src_pad)


def _scatter_body(msg_hbm, dst3_hbm, zeros_hbm, out_hbm, idx_v, rows_v, accum, sem):
    c = lax.axis_index("c")
    s = lax.axis_index("s")
    wid = s * 2 + c
    stripe = N // 16
    pltpu.sync_copy(zeros_hbm.at[pl.ds(s * stripe, stripe)],
                    accum.at[pl.ds(s * stripe, stripe)])
    pltpu.sync_copy(dst3_hbm.at[wid], idx_v)
    plsc.subcore_barrier()

    def body(g, carry):
        pltpu.sync_copy(msg_hbm.at[pl.ds(wid * EPW + g * GROUP, GROUP)], rows_v)
        cps = []
        for b in range(KG):
            cps.append(pltpu.async_copy(
                rows_v.at[pl.ds(b * CHUNK, CHUNK)],
                accum.at[idx_v.at[g * KG + b]], sem, add=True))
        for cp in cps:
            cp.wait()
        return carry

    lax.fori_loop(0, NGROUP, body, 0)
    plsc.subcore_barrier()
    pltpu.sync_copy(accum.at[pl.ds(s * stripe, stripe)],
                    out_hbm.at[c, pl.ds(s * stripe, stripe)])


def _sc_scatter(msg, dst3, zeros):
    return pl.kernel(
        _scatter_body,
        out_type=jax.ShapeDtypeStruct((2, N, 32), jnp.float32),
        mesh=_mesh,
        compiler_params=pltpu.CompilerParams(use_tc_tiling_on_sc=False),
        scratch_types=[
            pltpu.VMEM((NCHUNK, CHUNK), jnp.int32),
            pltpu.VMEM((GROUP, 32), jnp.float32),
            pltpu.VMEM_SHARED((N, 32), jnp.float32),
            pltpu.SemaphoreType.DMA,
        ],
    )(msg, dst3, zeros)


def _edge_body(ea_ref, xj_ref, en1_wT, en1_b, en2_wT, rp32, s32, b232, out_ref):
    ea = ea_ref[...]
    xj = xj_ref[...]
    h = jnp.maximum(ea @ en1_wT[...] + en1_b[...], 0.0)
    w = lax.dot(h.astype(jnp.bfloat16), en2_wT[...],
                preferred_element_type=jnp.float32)
    xjb = xj.astype(jnp.bfloat16)
    p = lax.dot(xjb, rp32[...], preferred_element_type=jnp.float32) * w
    msg = (lax.dot(p.astype(jnp.bfloat16), s32[...],
                   preferred_element_type=jnp.float32)
           + xj @ b232[...])
    lane = lax.broadcasted_iota(jnp.int32, (EB, 32), 1)
    msg = msg + jnp.where(lane == 24, 1.0, 0.0)
    row = lax.broadcasted_iota(jnp.int32, (EB, 32), 0) + pl.program_id(0) * EB
    out_ref[...] = jnp.where(row < E, msg, 0.0)


def _edge_msg(ea_pad, xj, en1_wT, en1_b, en2_wT, rp32, s32, b232):
    return pl.pallas_call(
        _edge_body,
        grid=(E_PAD // EB,),
        in_specs=[
            pl.BlockSpec((EB, EF), lambda i: (i, 0)),
            pl.BlockSpec((EB, 32), lambda i: (i, 0)),
            pl.BlockSpec((EF, EH), lambda i: (0, 0)),
            pl.BlockSpec((1, EH), lambda i: (0, 0)),
            pl.BlockSpec((EH, 576), lambda i: (0, 0)),
            pl.BlockSpec((32, 576), lambda i: (0, 0)),
            pl.BlockSpec((576, 32), lambda i: (0, 0)),
            pl.BlockSpec((32, 32), lambda i: (0, 0)),
        ],
        out_specs=pl.BlockSpec((EB, 32), lambda i: (i, 0)),
        out_shape=jax.ShapeDtypeStruct((E_PAD, 32), jnp.float32),
    )(ea_pad, xj, en1_wT, en1_b, en2_wT, rp32, s32, b232)


def _node_body(do_relu, parts_ref, x_ref, rw_ref, cb_ref, out_ref):
    s = parts_ref[0] + parts_ref[1]
    cnt = jnp.maximum(s[:, 24:25], 1.0)
    y = s / cnt + x_ref[...] @ rw_ref[...] + cb_ref[...]
    lane = lax.broadcasted_iota(jnp.int32, (N, 32), 1)
    y = jnp.where(lane < 24, y, 0.0)
    if do_relu:
        y = jnp.maximum(y, 0.0)
    out_ref[...] = y


def _node_update(parts, x32, rw32, cb32, do_relu):
    return pl.pallas_call(
        functools.partial(_node_body, do_relu),
        out_shape=jax.ShapeDtypeStruct((N, 32), jnp.float32),
    )(parts, x32, rw32, cb32)


def _s2s_body(parts_ref, x1_ref, rw_ref, cb_ref, batch_ref,
              wih_ref, whh_ref, bb_ref, fc2_ref, fc2b_ref, fc3_ref, fc3b_ref,
              out_ref):
    s = parts_ref[0] + parts_ref[1]
    cnt = jnp.maximum(s[:, 24:25], 1.0)
    lane = lax.broadcasted_iota(jnp.int32, (N, 32), 1)
    x2 = s / cnt + x1_ref[...] @ rw_ref[...] + cb_ref[...]
    x2 = jnp.where(lane < 24, x2, 0.0)

    at = (batch_ref[...] == lax.broadcasted_iota(jnp.int32, (N, NG), 1)).astype(jnp.float32)
    h = jnp.zeros((NG, 32), jnp.float32)
    c = jnp.zeros((NG, 32), jnp.float32)
    qs = jnp.zeros((NG, 64), jnp.float32)
    wih = wih_ref[...]
    whh = whh_ref[...]
    bb = bb_ref[...]
    for _ in range(3):
        g = qs @ wih + h @ whh + bb
        gi = jax.nn.sigmoid(g[:, 0:32])
        gf = jax.nn.sigmoid(g[:, 32:64])
        gg = jnp.tanh(g[:, 64:96])
        go = jax.nn.sigmoid(g[:, 96:128])
        c = gf * c + gi * gg
        h = go * jnp.tanh(c)
        glane = lax.broadcasted_iota(jnp.int32, (NG, 32), 1)
        q = jnp.where(glane < 24, h, 0.0)
        e = jnp.sum(x2 * (at @ q), axis=1, keepdims=True)
        e2 = jnp.where(at > 0.0, e, -1e30)
        m = jnp.max(e2, axis=0, keepdims=True)
        m_g = jnp.sum(at * m, axis=1, keepdims=True)
        a = jnp.exp(e - m_g)
        asum = lax.dot_general(a, at, (((0,), (0,)), ((), ())))
        asum_g = jnp.sum(at * asum, axis=1, keepdims=True)
        a = a / (asum_g + 1e-16)
        r = lax.dot_general(at, a * x2, (((0,), (0,)), ((), ())))
        qs = jnp.concatenate([q, r], axis=1)
    z = jnp.maximum(qs @ fc2_ref[...] + fc2b_ref[...], 0.0)
    out_ref[...] = z @ fc3_ref[...] + fc3b_ref[...]


def _set2set(parts2, x1, rw32, cb32, batch_col, wih_p, whh_p, bb_p, fc2_p, fc2b, fc3_p, fc3b):
    return pl.pallas_call(
        _s2s_body,
        out_shape=jax.ShapeDtypeStruct((NG, 2), jnp.float32),
    )(parts2, x1, rw32, cb32, batch_col, wih_p, whh_p, bb_p, fc2_p, fc2b, fc3_p, fc3b)


def _prep_consts(en1_w, en2_w, en2_b, root_w, conv_b, lstm_wih, lstm_whh,
                 lstm_bih, lstm_bhh, fc2_w, fc2_b, fc3_w, fc3_b):
    rp32 = jnp.zeros((32, 576)).at[:F].set(
        jnp.repeat(jnp.eye(F), F, axis=1)).astype(jnp.bfloat16)
    s32 = jnp.zeros((576, 32)).at[:, :F].set(
        jnp.tile(jnp.eye(F), (F, 1))).astype(jnp.bfloat16)
    b232 = jnp.zeros((32, 32)).at[:F, :F].set(en2_b.reshape(F, F))
    rw32 = jnp.zeros((32, 32)).at[:F, :F].set(root_w)
    cb32 = jnp.zeros((1, 32)).at[0, :F].set(conv_b)
    wih_p = jnp.zeros((64, 128))
    whh_p = jnp.zeros((32, 128))
    bb_p = jnp.zeros((1, 128))
    for gi in range(4):
        wg = lstm_wih[gi * F:(gi + 1) * F]
        wih_p = wih_p.at[0:F, gi * 32:gi * 32 + F].set(wg[:, :F].T)
        wih_p = wih_p.at[32:32 + F, gi * 32:gi * 32 + F].set(wg[:, F:].T)
        whh_p = whh_p.at[0:F, gi * 32:gi * 32 + F].set(lstm_whh[gi * F:(gi + 1) * F].T)
        bb_p = bb_p.at[0, gi * 32:gi * 32 + F].set(
            lstm_bih[gi * F:(gi + 1) * F] + lstm_bhh[gi * F:(gi + 1) * F])
    fc2_p = jnp.zeros((64, 8)).at[0:F].set(fc2_w[:, :F].T).at[32:32 + F].set(fc2_w[:, F:].T)
    return dict(en1_wT=en1_w.T, en2_wT=en2_w.T.astype(jnp.bfloat16),
                rp32=rp32, s32=s32, b232=b232,
                rw32=rw32, cb32=cb32, wih_p=wih_p, whh_p=whh_p, bb_p=bb_p,
                fc2_p=fc2_p, fc2b=fc2_b.reshape(1, 8), fc3_p=fc3_w.T,
                fc3b=fc3_b.reshape(1, 2))


def kernel(x, edge_index, edge_attr, batch, en1_w, en1_b, en2_w, en2_b, root_w, conv_b,
           lstm_wih, lstm_whh, lstm_bih, lstm_bhh, fc2_w, fc2_b, fc3_w, fc3_b):
    C = _prep_consts(en1_w, en2_w, en2_b, root_w, conv_b, lstm_wih, lstm_whh,
                     lstm_bih, lstm_bhh, fc2_w, fc2_b, fc3_w, fc3_b)
    en1_b_r = en1_b.reshape(1, EH)
    src = jnp.pad(edge_index[0], (0, E_PAD - E))
    dst3 = jnp.pad(edge_index[1], (0, E_PAD - E)).reshape(NW, NCHUNK, CHUNK)
    ea_pad = jnp.pad(edge_attr, ((0, E_PAD - E), (0, 0)))
    x32 = jnp.pad(x, ((0, 0), (0, 8)))
    batch_col = batch.reshape(N, 1)
    zeros = jnp.zeros((N, 32), jnp.float32)

    def layer(xc):
        xj = _sc_gather(xc, src)
        msg = _edge_msg(ea_pad, xj, C["en1_wT"], en1_b_r, C["en2_wT"],
                        C["rp32"], C["s32"], C["b232"])
        return _sc_scatter(msg, dst3, zeros)

    parts1 = layer(x32)
    x1 = _node_update(parts1, x32, C["rw32"], C["cb32"], True)
    parts2 = layer(x1)
    return _set2set(parts2, x1, C["rw32"], C["cb32"], batch_col,
                    C["wih_p"], C["whh_p"], C["bb_p"], C["fc2_p"], C["fc2b"],
                    C["fc3_p"], C["fc3b"])


# speedup vs baseline: 4.2684x; 1.1360x over previous
import functools
import jax, jax.numpy as jnp
from jax import lax
from jax.experimental import pallas as pl
from jax.experimental.pallas import tpu as pltpu, tpu_sc as plsc

N, E, NG, F, EF, EH = 10000, 160000, 64, 24, 16, 32
NW = 32
EPW = E // NW
CHUNK = 200
NCHUNK = EPW // CHUNK
EB = 1600

_mesh = plsc.VectorSubcoreMesh(core_axis_name="c", subcore_axis_name="s")


KG = 5
GROUP = KG * CHUNK
NGROUP = EPW // GROUP


def _gather_body(x_hbm, src_hbm, out_hbm, idx_v, bufs, gsem, wsem):
    wid = lax.axis_index("s") * 2 + lax.axis_index("c")
    base = wid * EPW
    pltpu.sync_copy(src_hbm.at[pl.ds(base, EPW)], idx_v)

    def issue(g):
        off = g * GROUP
        buf = bufs.at[g % 2]
        return [pltpu.async_copy(
            x_hbm.at[idx_v.at[pl.ds(off + b * CHUNK, CHUNK)]],
            buf.at[pl.ds(b * CHUNK, CHUNK)], gsem) for b in range(KG)]

    gcps = {0: issue(0)}
    wcps = {}
    for g in range(NGROUP):
        for cp in gcps[g]:
            cp.wait()
        wcps[g] = pltpu.async_copy(
            bufs.at[g % 2], out_hbm.at[pl.ds(base + g * GROUP, GROUP)], wsem)
        if g + 1 < NGROUP:
            if g >= 1:
                wcps[g - 1].wait()
            gcps[g + 1] = issue(g + 1)
    wcps[NGROUP - 2].wait()
    wcps[NGROUP - 1].wait()


def _sc_gather(x32, src):
    return pl.kernel(
        _gather_body,
        out_type=jax.ShapeDtypeStruct((E, 32), jnp.float32),
        mesh=_mesh,
        compiler_params=pltpu.CompilerParams(use_tc_tiling_on_sc=False),
        scratch_types=[
            pltpu.VMEM((EPW,), jnp.int32),
            pltpu.VMEM((2, GROUP, 32), jnp.float32),
            pltpu.SemaphoreType.DMA,
            pltpu.SemaphoreType.DMA,
        ],
    )(x32, src)


def _scatter_body(msg_hbm, dst3_hbm, zeros_hbm, out_hbm, idx_v, rows_v, accum, sem):
    c = lax.axis_index("c")
    s = lax.axis_index("s")
    wid = s * 2 + c
    stripe = N // 16
    pltpu.sync_copy(zeros_hbm.at[pl.ds(s * stripe, stripe)],
                    accum.at[pl.ds(s * stripe, stripe)])
    pltpu.sync_copy(dst3_hbm.at[wid], idx_v)
    plsc.subcore_barrier()

    def body(g, carry):
        pltpu.sync_copy(msg_hbm.at[pl.ds(wid * EPW + g * GROUP, GROUP)], rows_v)
        cps = []
        for b in range(KG):
            cps.append(pltpu.async_copy(
                rows_v.at[pl.ds(b * CHUNK, CHUNK)],
                accum.at[idx_v.at[g * KG + b]], sem, add=True))
        for cp in cps:
            cp.wait()
        return carry

    lax.fori_loop(0, NGROUP, body, 0)
    plsc.subcore_barrier()
    pltpu.sync_copy(accum.at[pl.ds(s * stripe, stripe)],
                    out_hbm.at[c, pl.ds(s * stripe, stripe)])


def _sc_scatter(msg, dst3, zeros):
    return pl.kernel(
        _scatter_body,
        out_type=jax.ShapeDtypeStruct((2, N, 32), jnp.float32),
        mesh=_mesh,
        compiler_params=pltpu.CompilerParams(use_tc_tiling_on_sc=False),
        scratch_types=[
            pltpu.VMEM((NCHUNK, CHUNK), jnp.int32),
            pltpu.VMEM((GROUP, 32), jnp.float32),
            pltpu.VMEM_SHARED((N, 32), jnp.float32),
            pltpu.SemaphoreType.DMA,
        ],
    )(msg, dst3, zeros)


def _edge_body(ea_ref, xj_ref, en1_wT, en1_b, en2_wT, rp32, s32, b232, out_ref):
    ea = ea_ref[...]
    xj = xj_ref[...]
    h = jnp.maximum(ea @ en1_wT[...] + en1_b[...], 0.0)
    w = lax.dot(h.astype(jnp.bfloat16), en2_wT[...],
                preferred_element_type=jnp.float32)
    xjb = xj.astype(jnp.bfloat16)
    p = lax.dot(xjb, rp32[...], preferred_element_type=jnp.float32) * w
    msg = (lax.dot(p.astype(jnp.bfloat16), s32[...],
                   preferred_element_type=jnp.float32)
           + xj @ b232[...])
    lane = lax.broadcasted_iota(jnp.int32, (EB, 32), 1)
    out_ref[...] = msg + jnp.where(lane == 24, 1.0, 0.0)


def _edge_msg(ea, xj, en1_wT, en1_b, en2_wT, rp32, s32, b232):
    return pl.pallas_call(
        _edge_body,
        grid=(E // EB,),
        in_specs=[
            pl.BlockSpec((EB, EF), lambda i: (i, 0)),
            pl.BlockSpec((EB, 32), lambda i: (i, 0)),
            pl.BlockSpec((EF, EH), lambda i: (0, 0)),
            pl.BlockSpec((1, EH), lambda i: (0, 0)),
            pl.BlockSpec((EH, 576), lambda i: (0, 0)),
            pl.BlockSpec((32, 576), lambda i: (0, 0)),
            pl.BlockSpec((576, 32), lambda i: (0, 0)),
            pl.BlockSpec((32, 32), lambda i: (0, 0)),
        ],
        out_specs=pl.BlockSpec((EB, 32), lambda i: (i, 0)),
        out_shape=jax.ShapeDtypeStruct((E, 32), jnp.float32),
    )(ea, xj, en1_wT, en1_b, en2_wT, rp32, s32, b232)


def _node_body(do_relu, parts_ref, x_ref, rw_ref, cb_ref, out_ref):
    s = parts_ref[0] + parts_ref[1]
    cnt = jnp.maximum(s[:, 24:25], 1.0)
    y = s / cnt + x_ref[...] @ rw_ref[...] + cb_ref[...]
    lane = lax.broadcasted_iota(jnp.int32, (N, 32), 1)
    y = jnp.where(lane < 24, y, 0.0)
    if do_relu:
        y = jnp.maximum(y, 0.0)
    out_ref[...] = y


def _node_update(parts, x32, rw32, cb32, do_relu):
    return pl.pallas_call(
        functools.partial(_node_body, do_relu),
        out_shape=jax.ShapeDtypeStruct((N, 32), jnp.float32),
    )(parts, x32, rw32, cb32)


def _s2s_body(parts_ref, x1_ref, rw_ref, cb_ref, batch_ref,
              wih_ref, whh_ref, bb_ref, fc2_ref, fc2b_ref, fc3_ref, fc3b_ref,
              out_ref):
    s = parts_ref[0] + parts_ref[1]
    cnt = jnp.maximum(s[:, 24:25], 1.0)
    lane = lax.broadcasted_iota(jnp.int32, (N, 32), 1)
    x2 = s / cnt + x1_ref[...] @ rw_ref[...] + cb_ref[...]
    x2 = jnp.where(lane < 24, x2, 0.0)

    at = (batch_ref[...] == lax.broadcasted_iota(jnp.int32, (N, NG), 1)).astype(jnp.float32)
    h = jnp.zeros((NG, 32), jnp.float32)
    c = jnp.zeros((NG, 32), jnp.float32)
    qs = jnp.zeros((NG, 64), jnp.float32)
    wih = wih_ref[...]
    whh = whh_ref[...]
    bb = bb_ref[...]
    for _ in range(3):
        g = qs @ wih + h @ whh + bb
        gi = jax.nn.sigmoid(g[:, 0:32])
        gf = jax.nn.sigmoid(g[:, 32:64])
        gg = jnp.tanh(g[:, 64:96])
        go = jax.nn.sigmoid(g[:, 96:128])
        c = gf * c + gi * gg
        h = go * jnp.tanh(c)
        glane = lax.broadcasted_iota(jnp.int32, (NG, 32), 1)
        q = jnp.where(glane < 24, h, 0.0)
        e = jnp.sum(x2 * (at @ q), axis=1, keepdims=True)
        e2 = jnp.where(at > 0.0, e, -1e30)
        m = jnp.max(e2, axis=0, keepdims=True)
        m_g = jnp.sum(at * m, axis=1, keepdims=True)
        a = jnp.exp(e - m_g)
        asum = lax.dot_general(a, at, (((0,), (0,)), ((), ())))
        asum_g = jnp.sum(at * asum, axis=1, keepdims=True)
        a = a / (asum_g + 1e-16)
        r = lax.dot_general(at, a * x2, (((0,), (0,)), ((), ())))
        qs = jnp.concatenate([q, r], axis=1)
    z = jnp.maximum(qs @ fc2_ref[...] + fc2b_ref[...], 0.0)
    out_ref[...] = z @ fc3_ref[...] + fc3b_ref[...]


def _set2set(parts2, x1, rw32, cb32, batch_col, wih_p, whh_p, bb_p, fc2_p, fc2b, fc3_p, fc3b):
    return pl.pallas_call(
        _s2s_body,
        out_shape=jax.ShapeDtypeStruct((NG, 2), jnp.float32),
    )(parts2, x1, rw32, cb32, batch_col, wih_p, whh_p, bb_p, fc2_p, fc2b, fc3_p, fc3b)


def _prep_consts(en1_w, en2_w, en2_b, root_w, conv_b, lstm_wih, lstm_whh,
                 lstm_bih, lstm_bhh, fc2_w, fc2_b, fc3_w, fc3_b):
    rp32 = jnp.zeros((32, 576)).at[:F].set(
        jnp.repeat(jnp.eye(F), F, axis=1)).astype(jnp.bfloat16)
    s32 = jnp.zeros((576, 32)).at[:, :F].set(
        jnp.tile(jnp.eye(F), (F, 1))).astype(jnp.bfloat16)
    b232 = jnp.zeros((32, 32)).at[:F, :F].set(en2_b.reshape(F, F))
    rw32 = jnp.zeros((32, 32)).at[:F, :F].set(root_w)
    cb32 = jnp.zeros((1, 32)).at[0, :F].set(conv_b)
    wih_p = jnp.zeros((64, 128))
    whh_p = jnp.zeros((32, 128))
    bb_p = jnp.zeros((1, 128))
    for gi in range(4):
        wg = lstm_wih[gi * F:(gi + 1) * F]
        wih_p = wih_p.at[0:F, gi * 32:gi * 32 + F].set(wg[:, :F].T)
        wih_p = wih_p.at[32:32 + F, gi * 32:gi * 32 + F].set(wg[:, F:].T)
        whh_p = whh_p.at[0:F, gi * 32:gi * 32 + F].set(lstm_whh[gi * F:(gi + 1) * F].T)
        bb_p = bb_p.at[0, gi * 32:gi * 32 + F].set(
            lstm_bih[gi * F:(gi + 1) * F] + lstm_bhh[gi * F:(gi + 1) * F])
    fc2_p = jnp.zeros((64, 8)).at[0:F].set(fc2_w[:, :F].T).at[32:32 + F].set(fc2_w[:, F:].T)
    return dict(en1_wT=en1_w.T, en2_wT=en2_w.T.astype(jnp.bfloat16),
                rp32=rp32, s32=s32, b232=b232,
                rw32=rw32, cb32=cb32, wih_p=wih_p, whh_p=whh_p, bb_p=bb_p,
                fc2_p=fc2_p, fc2b=fc2_b.reshape(1, 8), fc3_p=fc3_w.T,
                fc3b=fc3_b.reshape(1, 2))


def kernel(x, edge_index, edge_attr, batch, en1_w, en1_b, en2_w, en2_b, root_w, conv_b,
           lstm_wih, lstm_whh, lstm_bih, lstm_bhh, fc2_w, fc2_b, fc3_w, fc3_b):
    C = _prep_consts(en1_w, en2_w, en2_b, root_w, conv_b, lstm_wih, lstm_whh,
                     lstm_bih, lstm_bhh, fc2_w, fc2_b, fc3_w, fc3_b)
    en1_b_r = en1_b.reshape(1, EH)
    src = edge_index[0]
    dst3 = edge_index[1].reshape(NW, NCHUNK, CHUNK)
    x32 = jnp.pad(x, ((0, 0), (0, 8)))
    batch_col = batch.reshape(N, 1)
    zeros = jnp.zeros((N, 32), jnp.float32)

    def layer(xc):
        xj = _sc_gather(xc, src)
        msg = _edge_msg(edge_attr, xj, C["en1_wT"], en1_b_r, C["en2_wT"],
                        C["rp32"], C["s32"], C["b232"])
        return _sc_scatter(msg, dst3, zeros)

    parts1 = layer(x32)
    x1 = _node_update(parts1, x32, C["rw32"], C["cb32"], True)
    parts2 = layer(x1)
    return _set2set(parts2, x1, C["rw32"], C["cb32"], batch_col,
                    C["wih_p"], C["whh_p"], C["bb_p"], C["fc2_p"], C["fc2b"],
                    C["fc3_p"], C["fc3b"])


# speedup vs baseline: 5.3251x; 1.2476x over previous
import functools
import jax, jax.numpy as jnp
from jax import lax
from jax.experimental import pallas as pl
from jax.experimental.pallas import tpu as pltpu, tpu_sc as plsc

N, E, NG, F, EF, EH = 10000, 160000, 64, 24, 16, 32
NW = 32
EPW = E // NW
CHUNK = 200
NCHUNK = EPW // CHUNK
E4 = E // 4
EB4 = 2000
NQB = E4 // EB4

_mesh = plsc.VectorSubcoreMesh(core_axis_name="c", subcore_axis_name="s")


KG = 5
GROUP = KG * CHUNK
NGROUP = EPW // GROUP


def _gather_body(x_hbm, src_hbm, out_hbm, idx_v, bufs, gsem, wsem):
    wid = lax.axis_index("s") * 2 + lax.axis_index("c")
    q = wid // 8
    rb = (wid % 8) * EPW
    pltpu.sync_copy(src_hbm.at[pl.ds(wid * EPW, EPW)], idx_v)

    def issue(g):
        off = g * GROUP
        buf = bufs.at[g % 2]
        return [pltpu.async_copy(
            x_hbm.at[idx_v.at[pl.ds(off + b * CHUNK, CHUNK)]],
            buf.at[pl.ds(b * CHUNK, CHUNK)], gsem) for b in range(KG)]

    gcps = {0: issue(0)}
    wcps = {}
    for g in range(NGROUP):
        for cp in gcps[g]:
            cp.wait()
        wcps[g] = pltpu.async_copy(
            bufs.at[g % 2],
            out_hbm.at[pl.ds(rb + g * GROUP, GROUP), pl.ds(q * 32, 32)], wsem)
        if g + 1 < NGROUP:
            if g >= 1:
                wcps[g - 1].wait()
            gcps[g + 1] = issue(g + 1)
    wcps[NGROUP - 2].wait()
    wcps[NGROUP - 1].wait()


def _sc_gather(x32, src):
    return pl.kernel(
        _gather_body,
        out_type=jax.ShapeDtypeStruct((E4, 128), jnp.float32),
        mesh=_mesh,
        compiler_params=pltpu.CompilerParams(use_tc_tiling_on_sc=False),
        scratch_types=[
            pltpu.VMEM((EPW,), jnp.int32),
            pltpu.VMEM((2, GROUP, 32), jnp.float32),
            pltpu.SemaphoreType.DMA,
            pltpu.SemaphoreType.DMA,
        ],
    )(x32, src)


def _scatter_body(msg_hbm, dst3_hbm, zeros_hbm, out_hbm, idx_v, rows_v, accum, sem):
    c = lax.axis_index("c")
    s = lax.axis_index("s")
    wid = s * 2 + c
    q = wid // 8
    rb = (wid % 8) * EPW
    stripe = N // 16
    pltpu.sync_copy(zeros_hbm.at[pl.ds(s * stripe, stripe)],
                    accum.at[pl.ds(s * stripe, stripe)])
    pltpu.sync_copy(dst3_hbm.at[wid], idx_v)
    plsc.subcore_barrier()

    def body(g, carry):
        pltpu.sync_copy(
            msg_hbm.at[pl.ds(rb + g * GROUP, GROUP), pl.ds(q * 32, 32)], rows_v)
        cps = []
        for b in range(KG):
            cps.append(pltpu.async_copy(
                rows_v.at[pl.ds(b * CHUNK, CHUNK)],
                accum.at[idx_v.at[g * KG + b]], sem, add=True))
        for cp in cps:
            cp.wait()
        return carry

    lax.fori_loop(0, NGROUP, body, 0)
    plsc.subcore_barrier()
    pltpu.sync_copy(accum.at[pl.ds(s * stripe, stripe)],
                    out_hbm.at[c, pl.ds(s * stripe, stripe)])


def _sc_scatter(msg, dst3, zeros):
    return pl.kernel(
        _scatter_body,
        out_type=jax.ShapeDtypeStruct((2, N, 32), jnp.float32),
        mesh=_mesh,
        compiler_params=pltpu.CompilerParams(use_tc_tiling_on_sc=False),
        scratch_types=[
            pltpu.VMEM((NCHUNK, CHUNK), jnp.int32),
            pltpu.VMEM((GROUP, 32), jnp.float32),
            pltpu.VMEM_SHARED((N, 32), jnp.float32),
            pltpu.SemaphoreType.DMA,
        ],
    )(msg, dst3, zeros)


def _edge_body(ea0_ref, ea1_ref, ea2_ref, ea3_ref, xj_ref,
               en1_wT, en1_b, en2_wT, rp32, s32, b232, out_ref):
    ea_refs = (ea0_ref, ea1_ref, ea2_ref, ea3_ref)
    lane = lax.broadcasted_iota(jnp.int32, (EB4, 32), 1)
    count_lane = jnp.where(lane == 24, 1.0, 0.0)
    for k in range(4):
        ea = ea_refs[k][...]
        xj = xj_ref[:, k * 32:(k + 1) * 32]
        h = jnp.maximum(ea @ en1_wT[...] + en1_b[...], 0.0)
        w = lax.dot(h.astype(jnp.bfloat16), en2_wT[...],
                    preferred_element_type=jnp.float32)
        xjb = xj.astype(jnp.bfloat16)
        p = lax.dot(xjb, rp32[...], preferred_element_type=jnp.float32) * w
        msg = (lax.dot(p.astype(jnp.bfloat16), s32[...],
                       preferred_element_type=jnp.float32)
               + xj @ b232[...])
        out_ref[:, k * 32:(k + 1) * 32] = msg + count_lane


def _edge_msg(ea, xj, en1_wT, en1_b, en2_wT, rp32, s32, b232):
    ea_spec = [pl.BlockSpec((EB4, EF), functools.partial(
        lambda k, i: (k * NQB + i, 0), k)) for k in range(4)]
    return pl.pallas_call(
        _edge_body,
        grid=(NQB,),
        in_specs=ea_spec + [
            pl.BlockSpec((EB4, 128), lambda i: (i, 0)),
            pl.BlockSpec((EF, EH), lambda i: (0, 0)),
            pl.BlockSpec((1, EH), lambda i: (0, 0)),
            pl.BlockSpec((EH, 576), lambda i: (0, 0)),
            pl.BlockSpec((32, 576), lambda i: (0, 0)),
            pl.BlockSpec((576, 32), lambda i: (0, 0)),
            pl.BlockSpec((32, 32), lambda i: (0, 0)),
        ],
        out_specs=pl.BlockSpec((EB4, 128), lambda i: (i, 0)),
        out_shape=jax.ShapeDtypeStruct((E4, 128), jnp.float32),
    )(ea, ea, ea, ea, xj, en1_wT, en1_b, en2_wT, rp32, s32, b232)


def _node_body(do_relu, parts_ref, x_ref, rw_ref, cb_ref, out_ref):
    s = parts_ref[0] + parts_ref[1]
    cnt = jnp.maximum(s[:, 24:25], 1.0)
    y = s / cnt + x_ref[...] @ rw_ref[...] + cb_ref[...]
    lane = lax.broadcasted_iota(jnp.int32, (N, 32), 1)
    y = jnp.where(lane < 24, y, 0.0)
    if do_relu:
        y = jnp.maximum(y, 0.0)
    out_ref[...] = y


def _node_update(parts, x32, rw32, cb32, do_relu):
    return pl.pallas_call(
        functools.partial(_node_body, do_relu),
        out_shape=jax.ShapeDtypeStruct((N, 32), jnp.float32),
    )(parts, x32, rw32, cb32)


def _s2s_body(parts_ref, x1_ref, rw_ref, cb_ref, batch_ref,
              wih_ref, whh_ref, bb_ref, fc2_ref, fc2b_ref, fc3_ref, fc3b_ref,
              out_ref):
    s = parts_ref[0] + parts_ref[1]
    cnt = jnp.maximum(s[:, 24:25], 1.0)
    lane = lax.broadcasted_iota(jnp.int32, (N, 32), 1)
    x2 = s / cnt + x1_ref[...] @ rw_ref[...] + cb_ref[...]
    x2 = jnp.where(lane < 24, x2, 0.0)

    at = (batch_ref[...] == lax.broadcasted_iota(jnp.int32, (N, NG), 1)).astype(jnp.float32)
    h = jnp.zeros((NG, 32), jnp.float32)
    c = jnp.zeros((NG, 32), jnp.float32)
    qs = jnp.zeros((NG, 64), jnp.float32)
    wih = wih_ref[...]
    whh = whh_ref[...]
    bb = bb_ref[...]
    for _ in range(3):
        g = qs @ wih + h @ whh + bb
        gi = jax.nn.sigmoid(g[:, 0:32])
        gf = jax.nn.sigmoid(g[:, 32:64])
        gg = jnp.tanh(g[:, 64:96])
        go = jax.nn.sigmoid(g[:, 96:128])
        c = gf * c + gi * gg
        h = go * jnp.tanh(c)
        glane = lax.broadcasted_iota(jnp.int32, (NG, 32), 1)
        q = jnp.where(glane < 24, h, 0.0)
        e = jnp.sum(x2 * (at @ q), axis=1, keepdims=True)
        e2 = jnp.where(at > 0.0, e, -1e30)
        m = jnp.max(e2, axis=0, keepdims=True)
        m_g = jnp.sum(at * m, axis=1, keepdims=True)
        a = jnp.exp(e - m_g)
        asum = lax.dot_general(a, at, (((0,), (0,)), ((), ())))
        asum_g = jnp.sum(at * asum, axis=1, keepdims=True)
        a = a / (asum_g + 1e-16)
        r = lax.dot_general(at, a * x2, (((0,), (0,)), ((), ())))
        qs = jnp.concatenate([q, r], axis=1)
    z = jnp.maximum(qs @ fc2_ref[...] + fc2b_ref[...], 0.0)
    out_ref[...] = z @ fc3_ref[...] + fc3b_ref[...]


def _set2set(parts2, x1, rw32, cb32, batch_col, wih_p, whh_p, bb_p, fc2_p, fc2b, fc3_p, fc3b):
    return pl.pallas_call(
        _s2s_body,
        out_shape=jax.ShapeDtypeStruct((NG, 2), jnp.float32),
    )(parts2, x1, rw32, cb32, batch_col, wih_p, whh_p, bb_p, fc2_p, fc2b, fc3_p, fc3b)


def _prep_consts(en1_w, en2_w, en2_b, root_w, conv_b, lstm_wih, lstm_whh,
                 lstm_bih, lstm_bhh, fc2_w, fc2_b, fc3_w, fc3_b):
    rp32 = jnp.zeros((32, 576)).at[:F].set(
        jnp.repeat(jnp.eye(F), F, axis=1)).astype(jnp.bfloat16)
    s32 = jnp.zeros((576, 32)).at[:, :F].set(
        jnp.tile(jnp.eye(F), (F, 1))).astype(jnp.bfloat16)
    b232 = jnp.zeros((32, 32)).at[:F, :F].set(en2_b.reshape(F, F))
    rw32 = jnp.zeros((32, 32)).at[:F, :F].set(root_w)
    cb32 = jnp.zeros((1, 32)).at[0, :F].set(conv_b)
    wih_p = jnp.zeros((64, 128))
    whh_p = jnp.zeros((32, 128))
    bb_p = jnp.zeros((1, 128))
    for gi in range(4):
        wg = lstm_wih[gi * F:(gi + 1) * F]
        wih_p = wih_p.at[0:F, gi * 32:gi * 32 + F].set(wg[:, :F].T)
        wih_p = wih_p.at[32:32 + F, gi * 32:gi * 32 + F].set(wg[:, F:].T)
        whh_p = whh_p.at[0:F, gi * 32:gi * 32 + F].set(lstm_whh[gi * F:(gi + 1) * F].T)
        bb_p = bb_p.at[0, gi * 32:gi * 32 + F].set(
            lstm_bih[gi * F:(gi + 1) * F] + lstm_bhh[gi * F:(gi + 1) * F])
    fc2_p = jnp.zeros((64, 8)).at[0:F].set(fc2_w[:, :F].T).at[32:32 + F].set(fc2_w[:, F:].T)
    return dict(en1_wT=en1_w.T, en2_wT=en2_w.T.astype(jnp.bfloat16),
                rp32=rp32, s32=s32, b232=b232,
                rw32=rw32, cb32=cb32, wih_p=wih_p, whh_p=whh_p, bb_p=bb_p,
                fc2_p=fc2_p, fc2b=fc2_b.reshape(1, 8), fc3_p=fc3_w.T,
                fc3b=fc3_b.reshape(1, 2))


def kernel(x, edge_index, edge_attr, batch, en1_w, en1_b, en2_w, en2_b, root_w, conv_b,
           lstm_wih, lstm_whh, lstm_bih, lstm_bhh, fc2_w, fc2_b, fc3_w, fc3_b):
    C = _prep_consts(en1_w, en2_w, en2_b, root_w, conv_b, lstm_wih, lstm_whh,
                     lstm_bih, lstm_bhh, fc2_w, fc2_b, fc3_w, fc3_b)
    en1_b_r = en1_b.reshape(1, EH)
    src = edge_index[0]
    dst3 = edge_index[1].reshape(NW, NCHUNK, CHUNK)
    x32 = jnp.pad(x, ((0, 0), (0, 8)))
    batch_col = batch.reshape(N, 1)
    zeros = jnp.zeros((N, 32), jnp.float32)

    def layer(xc):
        xj = _sc_gather(xc, src)
        msg = _edge_msg(edge_attr, xj, C["en1_wT"], en1_b_r, C["en2_wT"],
                        C["rp32"], C["s32"], C["b232"])
        return _sc_scatter(msg, dst3, zeros)

    parts1 = layer(x32)
    x1 = _node_update(parts1, x32, C["rw32"], C["cb32"], True)
    parts2 = layer(x1)
    return _set2set(parts2, x1, C["rw32"], C["cb32"], batch_col,
                    C["wih_p"], C["whh_p"], C["bb_p"], C["fc2_p"], C["fc2b"],
                    C["fc3_p"], C["fc3b"])
